# baseline (device time: 43862 ns/iter reference)
import jax
import jax.numpy as jnp
from jax import lax
from jax.experimental import pallas as pl
from jax.experimental.pallas import tpu as pltpu

B, S, H, Dh, Dr = 2, 256, 16, 64, 32
D = 1024
DC_SH = 64


def kernel(x, Wdkv, Wuk, Wuv, Wq, Wqr, Wkr, Wo):
    def body(x_ref, wdkv_ref, wuk_ref, wuv_ref, wq_ref, wqr_ref, wkr_ref,
             wo_ref, out_ref,
             c_send, c_recv, wuk_recv, wuv_recv, send_sems, recv_sems):
        my_x = lax.axis_index("x")
        my_y = lax.axis_index("y")
        my_z = lax.axis_index("z")
        partner = (my_x, 1 - my_y, my_z)

        barrier_sem = pltpu.get_barrier_semaphore()
        pl.semaphore_signal(barrier_sem, inc=1, device_id=partner,
                            device_id_type=pl.DeviceIdType.MESH)
        pl.semaphore_wait(barrier_sem, 1)

        x2d = x_ref[...].reshape(B * S, D)

        c_loc = jnp.dot(x2d, wdkv_ref[...], preferred_element_type=jnp.float32)
        c_send[...] = c_loc

        rdmas = []
        for i, (src, dst) in enumerate([
            (c_send, c_recv),
            (wuk_ref, wuk_recv),
            (wuv_ref, wuv_recv),
        ]):
            r = pltpu.make_async_remote_copy(
                src_ref=src, dst_ref=dst,
                send_sem=send_sems.at[i], recv_sem=recv_sems.at[i],
                device_id=partner, device_id_type=pl.DeviceIdType.MESH,
            )
            r.start()
            rdmas.append(r)

        q_all = jnp.dot(x2d, wq_ref[...], preferred_element_type=jnp.float32)
        qr_all = jnp.dot(x2d, wqr_ref[...], preferred_element_type=jnp.float32)
        kr_all = jnp.dot(x2d, wkr_ref[...], preferred_element_type=jnp.float32)

        for r in rdmas:
            r.wait()

        k_all = (jnp.dot(c_loc, wuk_ref[...], preferred_element_type=jnp.float32)
                 + jnp.dot(c_recv[...], wuk_recv[...],
                           preferred_element_type=jnp.float32))
        v_all = (jnp.dot(c_loc, wuv_ref[...], preferred_element_type=jnp.float32)
                 + jnp.dot(c_recv[...], wuv_recv[...],
                           preferred_element_type=jnp.float32))

        scale = (Dh + Dr) ** -0.5
        dn_t = (((1,), (1,)), ((), ()))
        for b in range(B):
            rows = slice(b * S, (b + 1) * S)
            kr_b = kr_all[rows, :]
            o_heads = []
            for h in range(H):
                q = q_all[rows, h * Dh:(h + 1) * Dh]
                k = k_all[rows, h * Dh:(h + 1) * Dh]
                v = v_all[rows, h * Dh:(h + 1) * Dh]
                qr = qr_all[rows, h * Dr:(h + 1) * Dr]
                s = (lax.dot_general(q, k, dn_t,
                                     preferred_element_type=jnp.float32)
                     + lax.dot_general(qr, kr_b, dn_t,
                                       preferred_element_type=jnp.float32))
                s = s * scale
                m = jnp.max(s, axis=-1, keepdims=True)
                p = jnp.exp(s - m)
                p = p / jnp.sum(p, axis=-1, keepdims=True)
                o_heads.append(jnp.dot(p, v, preferred_element_type=jnp.float32))
            o_b = jnp.concatenate(o_heads, axis=1)
            out_ref[b, :, :] = jnp.dot(o_b, wo_ref[...],
                                       preferred_element_type=jnp.float32)

    return pl.pallas_call(
        body,
        out_shape=jax.ShapeDtypeStruct((B, S, H * Dh), jnp.float32),
        in_specs=[pl.BlockSpec(memory_space=pltpu.VMEM)] * 8,
        out_specs=pl.BlockSpec(memory_space=pltpu.VMEM),
        scratch_shapes=[
            pltpu.VMEM((B * S, DC_SH), jnp.float32),
            pltpu.VMEM((B * S, DC_SH), jnp.float32),
            pltpu.VMEM((DC_SH, D), jnp.float32),
            pltpu.VMEM((DC_SH, D), jnp.float32),
            pltpu.SemaphoreType.DMA((3,)),
            pltpu.SemaphoreType.DMA((3,)),
        ],
        compiler_params=pltpu.CompilerParams(collective_id=0),
    )(x, Wdkv, Wuk, Wuv, Wq, Wqr, Wkr, Wo)


# device time: 38191 ns/iter; 1.1485x vs baseline; 1.1485x over previous
import jax
import jax.numpy as jnp
from jax import lax
from jax.experimental import pallas as pl
from jax.experimental.pallas import tpu as pltpu

B, S, H, Dh, Dr = 2, 256, 16, 64, 32
D = 1024
DC_SH = 64


def kernel(x, Wdkv, Wuk, Wuv, Wq, Wqr, Wkr, Wo):
    def body(x_ref, wdkv_ref, wuk_ref, wuv_ref, wq_ref, wqr_ref, wkr_ref,
             wo_ref, out_ref,
             c_send, c_recv, wuk_recv, wuv_recv, send_sems, recv_sems):
        my_x = lax.axis_index("x")
        my_y = lax.axis_index("y")
        my_z = lax.axis_index("z")
        partner = (my_x, 1 - my_y, my_z)

        barrier_sem = pltpu.get_barrier_semaphore()
        pl.semaphore_signal(barrier_sem, inc=1, device_id=partner,
                            device_id_type=pl.DeviceIdType.MESH)
        pl.semaphore_wait(barrier_sem, 1)

        x2d = x_ref[...].reshape(B * S, D)

        c_loc = jnp.dot(x2d, wdkv_ref[...], preferred_element_type=jnp.float32)
        c_send[...] = c_loc

        rdmas = []
        for i, (src, dst) in enumerate([
            (c_send, c_recv),
            (wuk_ref, wuk_recv),
            (wuv_ref, wuv_recv),
        ]):
            r = pltpu.make_async_remote_copy(
                src_ref=src, dst_ref=dst,
                send_sem=send_sems.at[i], recv_sem=recv_sems.at[i],
                device_id=partner, device_id_type=pl.DeviceIdType.MESH,
            )
            r.start()
            rdmas.append(r)

        q_all = jnp.dot(x2d, wq_ref[...], preferred_element_type=jnp.float32)
        qr_all = jnp.dot(x2d, wqr_ref[...], preferred_element_type=jnp.float32)
        kr_all = jnp.dot(x2d, wkr_ref[...], preferred_element_type=jnp.float32)

        for r in rdmas:
            r.wait()

        k_all = (jnp.dot(c_loc, wuk_ref[...], preferred_element_type=jnp.float32)
                 + jnp.dot(c_recv[...], wuk_recv[...],
                           preferred_element_type=jnp.float32))
        v_all = (jnp.dot(c_loc, wuv_ref[...], preferred_element_type=jnp.float32)
                 + jnp.dot(c_recv[...], wuv_recv[...],
                           preferred_element_type=jnp.float32))

        scale = (Dh + Dr) ** -0.5
        q3 = q_all.reshape(B, S, H, Dh).transpose(0, 2, 1, 3).reshape(B * H, S, Dh)
        k3 = k_all.reshape(B, S, H, Dh).transpose(0, 2, 1, 3).reshape(B * H, S, Dh)
        v3 = v_all.reshape(B, S, H, Dh).transpose(0, 2, 1, 3).reshape(B * H, S, Dh)
        qr3 = qr_all.reshape(B, S, H, Dr).transpose(0, 2, 1, 3).reshape(B * H, S, Dr)
        kr3 = jnp.broadcast_to(
            kr_all.reshape(B, 1, S, Dr), (B, H, S, Dr)
        ).reshape(B * H, S, Dr)

        dn_bt = (((2,), (2,)), ((0,), (0,)))
        s = lax.dot_general(q3, k3, dn_bt, preferred_element_type=jnp.float32)
        s = s + lax.dot_general(qr3, kr3, dn_bt,
                                preferred_element_type=jnp.float32)
        s = s * scale
        m = jnp.max(s, axis=-1, keepdims=True)
        p = jnp.exp(s - m)
        p = p / jnp.sum(p, axis=-1, keepdims=True)
        dn_pv = (((2,), (1,)), ((0,), (0,)))
        o3 = lax.dot_general(p, v3, dn_pv, preferred_element_type=jnp.float32)
        o2 = (o3.reshape(B, H, S, Dh).transpose(0, 2, 1, 3)
              .reshape(B * S, H * Dh))
        out = jnp.dot(o2, wo_ref[...], preferred_element_type=jnp.float32)
        out_ref[...] = out.reshape(B, S, H * Dh)

    return pl.pallas_call(
        body,
        out_shape=jax.ShapeDtypeStruct((B, S, H * Dh), jnp.float32),
        in_specs=[pl.BlockSpec(memory_space=pltpu.VMEM)] * 8,
        out_specs=pl.BlockSpec(memory_space=pltpu.VMEM),
        scratch_shapes=[
            pltpu.VMEM((B * S, DC_SH), jnp.float32),
            pltpu.VMEM((B * S, DC_SH), jnp.float32),
            pltpu.VMEM((DC_SH, D), jnp.float32),
            pltpu.VMEM((DC_SH, D), jnp.float32),
            pltpu.SemaphoreType.DMA((3,)),
            pltpu.SemaphoreType.DMA((3,)),
        ],
        compiler_params=pltpu.CompilerParams(collective_id=0),
    )(x, Wdkv, Wuk, Wuv, Wq, Wqr, Wkr, Wo)


# device time: 28754 ns/iter; 1.5254x vs baseline; 1.3282x over previous
import jax
import jax.numpy as jnp
from jax import lax
from jax.experimental import pallas as pl
from jax.experimental.pallas import tpu as pltpu

B, S, H, Dh, Dr = 2, 256, 16, 64, 32
D = 1024
DC_SH = 64
PACK_N = 2 * DC_SH


def kernel(x, Wdkv, Wuk, Wuv, Wq, Wqr, Wkr, Wo):
    def body(x_ref, wdkv_ref, wuk_ref, wuv_ref, wq_ref, wqr_ref, wkr_ref,
             wo_ref, out_ref,
             c_send, c_recv, pack_send, pack_recv, send_sems, recv_sems):
        my_x = lax.axis_index("x")
        my_y = lax.axis_index("y")
        my_z = lax.axis_index("z")
        partner = (my_x, 1 - my_y, my_z)

        barrier_sem = pltpu.get_barrier_semaphore()
        pl.semaphore_signal(barrier_sem, inc=1, device_id=partner,
                            device_id_type=pl.DeviceIdType.MESH)
        pl.semaphore_wait(barrier_sem, 1)

        bf16 = jnp.bfloat16

        def heads3(t, d):
            return (t.astype(bf16).reshape(B, S, H, d)
                    .transpose(0, 2, 1, 3).reshape(B * H, S, d))

        x2b = x_ref[...].reshape(B * S, D).astype(bf16)
        wdkv_b = wdkv_ref[...].astype(bf16)
        wuk_b = wuk_ref[...].astype(bf16)
        wuv_b = wuv_ref[...].astype(bf16)

        c_b = jnp.dot(x2b, wdkv_b,
                      preferred_element_type=jnp.float32).astype(bf16)
        c_send[...] = c_b
        pack_send[0:DC_SH, :] = wuk_b
        pack_send[DC_SH:PACK_N, :] = wuv_b

        rdmas = []
        for i, (src_r, dst_r) in enumerate([(c_send, c_recv),
                                            (pack_send, pack_recv)]):
            r = pltpu.make_async_remote_copy(
                src_ref=src_r, dst_ref=dst_r,
                send_sem=send_sems.at[i], recv_sem=recv_sems.at[i],
                device_id=partner, device_id_type=pl.DeviceIdType.MESH,
            )
            r.start()
            rdmas.append(r)

        q_all = jnp.dot(x2b, wq_ref[...].astype(bf16),
                        preferred_element_type=jnp.float32)
        qr_all = jnp.dot(x2b, wqr_ref[...].astype(bf16),
                         preferred_element_type=jnp.float32)
        kr_all = jnp.dot(x2b, wkr_ref[...].astype(bf16),
                         preferred_element_type=jnp.float32)

        q96 = jnp.concatenate(
            [q_all.astype(bf16).reshape(B, S, H, Dh),
             qr_all.astype(bf16).reshape(B, S, H, Dr)], axis=3
        ).transpose(0, 2, 1, 3).reshape(B * H, S, Dh + Dr)

        dn_bt = (((2,), (2,)), ((0,), (0,)))

        for r in rdmas:
            r.wait()

        DC = 2 * DC_SH
        c_full = jnp.concatenate([c_b, c_recv[...]], axis=1)
        c4 = jnp.broadcast_to(
            c_full.reshape(B, 1, S, DC), (B, H, S, DC)
        ).reshape(B * H, S, DC)
        wuk_full = jnp.concatenate([wuk_b, pack_recv[0:DC_SH, :]], axis=0)
        wuv_full = jnp.concatenate([wuv_b, pack_recv[DC_SH:PACK_N, :]],
                                   axis=0)

        def per_head_rhs(w):
            w3 = w.reshape(DC, H, Dh).transpose(1, 0, 2)
            return jnp.broadcast_to(
                w3.reshape(1, H, DC, Dh), (B, H, DC, Dh)
            ).reshape(B * H, DC, Dh)

        dn_kv = (((2,), (1,)), ((0,), (0,)))
        k3 = lax.dot_general(c4, per_head_rhs(wuk_full), dn_kv,
                             preferred_element_type=jnp.float32)
        v3 = lax.dot_general(c4, per_head_rhs(wuv_full), dn_kv,
                             preferred_element_type=jnp.float32).astype(bf16)
        kr3 = jnp.broadcast_to(
            kr_all.astype(bf16).reshape(B, 1, S, Dr), (B, H, S, Dr)
        ).reshape(B * H, S, Dr)
        k96 = jnp.concatenate([k3.astype(bf16), kr3], axis=2)

        scale = (Dh + Dr) ** -0.5
        s = lax.dot_general(q96, k96, dn_bt,
                            preferred_element_type=jnp.float32)
        p = jnp.exp(s * scale)
        denom = jnp.sum(p, axis=-1, keepdims=True)
        dn_pv = (((2,), (1,)), ((0,), (0,)))
        o3 = lax.dot_general(p.astype(bf16), v3, dn_pv,
                             preferred_element_type=jnp.float32)
        o3 = o3 / denom
        o2 = (o3.reshape(B, H, S, Dh).transpose(0, 2, 1, 3)
              .reshape(B * S, H * Dh)).astype(bf16)
        out = jnp.dot(o2, wo_ref[...].astype(bf16),
                      preferred_element_type=jnp.float32)
        out_ref[...] = out.reshape(B, S, H * Dh)

    return pl.pallas_call(
        body,
        out_shape=jax.ShapeDtypeStruct((B, S, H * Dh), jnp.float32),
        in_specs=[pl.BlockSpec(memory_space=pltpu.VMEM)] * 8,
        out_specs=pl.BlockSpec(memory_space=pltpu.VMEM),
        scratch_shapes=[
            pltpu.VMEM((B * S, DC_SH), jnp.bfloat16),
            pltpu.VMEM((B * S, DC_SH), jnp.bfloat16),
            pltpu.VMEM((PACK_N, D), jnp.bfloat16),
            pltpu.VMEM((PACK_N, D), jnp.bfloat16),
            pltpu.SemaphoreType.DMA((2,)),
            pltpu.SemaphoreType.DMA((2,)),
        ],
        compiler_params=pltpu.CompilerParams(collective_id=0),
    )(x, Wdkv, Wuk, Wuv, Wq, Wqr, Wkr, Wo)
